# baseline (device time: 59443 ns/iter reference)
import jax
import jax.numpy as jnp
from jax import lax
from jax.experimental import pallas as pl
from jax.experimental.pallas import tpu as pltpu

N_DEV = 16
WINDOW = 128


def kernel(x, Wq, K_ext, V_ext, Wo):
    B, Sq, D = x.shape
    _, Skv, H_loc, Dh = K_ext.shape
    d_loc = H_loc * Dh
    CH = Sq // N_DEV

    def body(x_ref, wq_ref, k_ref, v_ref, wo_ref, out_ref,
             wq_s, wo_s, q_s, ctx_s, acc_s, rs_buf,
             load_sems, rs_send_sem, rs_recv_sem, ag_send_sem, ag_recv_sem):
        my = lax.axis_index("i")

        rs_buf[...] = jnp.zeros_like(rs_buf)

        wq_dma = pltpu.make_async_copy(
            wq_ref.at[:, pl.ds(my * d_loc, d_loc)], wq_s, load_sems.at[0])
        wq_dma.start()
        wo_dma = pltpu.make_async_copy(
            wo_ref.at[pl.ds(my * d_loc, d_loc), :], wo_s, load_sems.at[1])
        wo_dma.start()

        bar = pltpu.get_barrier_semaphore()
        for j in range(N_DEV):
            pl.semaphore_signal(bar, inc=1, device_id=(j,),
                                device_id_type=pl.DeviceIdType.MESH)
        pl.semaphore_wait(bar, N_DEV)

        wq_dma.wait()
        wo_dma.wait()

        xb = x_ref[...].astype(jnp.bfloat16).reshape(B * Sq, D)
        q = jnp.dot(xb, wq_s[...].astype(jnp.bfloat16),
                    preferred_element_type=jnp.float32)
        q_s[...] = (q * 0.125).reshape(B, Sq, d_loc).astype(jnp.bfloat16)

        qi = lax.broadcasted_iota(jnp.int32, (Sq, Skv), 0)
        ki = lax.broadcasted_iota(jnp.int32, (Sq, Skv), 1)
        mask = jnp.abs(qi - ki) <= WINDOW

        for b in range(B):
            for h in range(H_loc):
                qbh = q_s[b, :, h * Dh:(h + 1) * Dh]
                kbh = k_ref[b, :, h, :].astype(jnp.bfloat16)
                s = lax.dot_general(
                    qbh, kbh, (((1,), (1,)), ((), ())),
                    preferred_element_type=jnp.float32)
                e = jnp.where(mask, jnp.exp(s), 0.0)
                denom = jnp.sum(e, axis=1, keepdims=True)
                vbh = v_ref[b, :, h, :].astype(jnp.bfloat16)
                ctx = jnp.dot(e.astype(jnp.bfloat16), vbh,
                              preferred_element_type=jnp.float32)
                ctx_s[b, :, h * Dh:(h + 1) * Dh] = (
                    ctx / denom).astype(jnp.bfloat16)

        part = jnp.dot(ctx_s[...].reshape(B * Sq, d_loc),
                       wo_s[...].astype(jnp.bfloat16),
                       preferred_element_type=jnp.float32)
        acc_s[...] = part.reshape(B, Sq, D).astype(jnp.bfloat16)

        for j in range(N_DEV):
            @pl.when(my != j)
            def _(j=j):
                pltpu.make_async_remote_copy(
                    src_ref=acc_s.at[:, pl.ds(j * CH, CH), :],
                    dst_ref=rs_buf.at[pl.ds(my * B, B)],
                    send_sem=rs_send_sem,
                    recv_sem=rs_recv_sem,
                    device_id=(j,),
                    device_id_type=pl.DeviceIdType.MESH,
                ).start()

        for j in range(N_DEV):
            @pl.when(my != j)
            def _(j=j):
                pltpu.make_async_remote_copy(
                    src_ref=acc_s.at[:, pl.ds(j * CH, CH), :],
                    dst_ref=rs_buf.at[pl.ds(j * B, B)],
                    send_sem=rs_send_sem,
                    recv_sem=rs_recv_sem,
                    device_id=(j,),
                    device_id_type=pl.DeviceIdType.MESH,
                ).wait_recv()

        own = acc_s[:, pl.ds(my * CH, CH), :].astype(jnp.float32)
        red = own + jnp.sum(
            rs_buf[...].reshape(N_DEV, B, CH, D).astype(jnp.float32), axis=0)
        out_ref[:, pl.ds(my * CH, CH), :] = red.astype(jnp.bfloat16)

        for j in range(N_DEV):
            @pl.when(my != j)
            def _(j=j):
                pltpu.make_async_remote_copy(
                    src_ref=acc_s.at[:, pl.ds(j * CH, CH), :],
                    dst_ref=rs_buf.at[pl.ds(j * B, B)],
                    send_sem=rs_send_sem,
                    recv_sem=rs_recv_sem,
                    device_id=(j,),
                    device_id_type=pl.DeviceIdType.MESH,
                ).wait_send()

        for j in range(N_DEV):
            @pl.when(my != j)
            def _(j=j):
                pltpu.make_async_remote_copy(
                    src_ref=out_ref.at[:, pl.ds(my * CH, CH), :],
                    dst_ref=out_ref.at[:, pl.ds(my * CH, CH), :],
                    send_sem=ag_send_sem,
                    recv_sem=ag_recv_sem,
                    device_id=(j,),
                    device_id_type=pl.DeviceIdType.MESH,
                ).start()

        for j in range(N_DEV):
            @pl.when(my != j)
            def _(j=j):
                pltpu.make_async_remote_copy(
                    src_ref=out_ref.at[:, pl.ds(j * CH, CH), :],
                    dst_ref=out_ref.at[:, pl.ds(j * CH, CH), :],
                    send_sem=ag_send_sem,
                    recv_sem=ag_recv_sem,
                    device_id=(j,),
                    device_id_type=pl.DeviceIdType.MESH,
                ).wait_recv()

        for j in range(N_DEV):
            @pl.when(my != j)
            def _(j=j):
                pltpu.make_async_remote_copy(
                    src_ref=out_ref.at[:, pl.ds(my * CH, CH), :],
                    dst_ref=out_ref.at[:, pl.ds(my * CH, CH), :],
                    send_sem=ag_send_sem,
                    recv_sem=ag_recv_sem,
                    device_id=(j,),
                    device_id_type=pl.DeviceIdType.MESH,
                ).wait_send()

    return pl.pallas_call(
        body,
        out_shape=jax.ShapeDtypeStruct((B, Sq, D), jnp.bfloat16),
        in_specs=[
            pl.BlockSpec(memory_space=pltpu.MemorySpace.VMEM),
            pl.BlockSpec(memory_space=pltpu.MemorySpace.HBM),
            pl.BlockSpec(memory_space=pltpu.MemorySpace.VMEM),
            pl.BlockSpec(memory_space=pltpu.MemorySpace.VMEM),
            pl.BlockSpec(memory_space=pltpu.MemorySpace.HBM),
        ],
        out_specs=pl.BlockSpec(memory_space=pltpu.MemorySpace.VMEM),
        scratch_shapes=[
            pltpu.VMEM((D, d_loc), jnp.float32),
            pltpu.VMEM((d_loc, D), jnp.float32),
            pltpu.VMEM((B, Sq, d_loc), jnp.bfloat16),
            pltpu.VMEM((B, Sq, d_loc), jnp.bfloat16),
            pltpu.VMEM((B, Sq, D), jnp.bfloat16),
            pltpu.VMEM((N_DEV * B, CH, D), jnp.bfloat16),
            pltpu.SemaphoreType.DMA((2,)),
            pltpu.SemaphoreType.DMA,
            pltpu.SemaphoreType.DMA,
            pltpu.SemaphoreType.DMA,
            pltpu.SemaphoreType.DMA,
        ],
        compiler_params=pltpu.CompilerParams(collective_id=0),
    )(x, Wq, K_ext, V_ext, Wo)


# device time: 53274 ns/iter; 1.1158x vs baseline; 1.1158x over previous
import jax
import jax.numpy as jnp
from jax import lax
from jax.experimental import pallas as pl
from jax.experimental.pallas import tpu as pltpu

N_DEV = 16
WINDOW = 128


def kernel(x, Wq, K_ext, V_ext, Wo):
    B, Sq, D = x.shape
    _, Skv, H_loc, Dh = K_ext.shape
    d_loc = H_loc * Dh
    CH = Sq // N_DEV

    def body(x_ref, wq_ref, k_ref, v_ref, wo_ref, out_ref,
             wq_s, wo_s, q_s, ctx_s, acc_s, rs_buf,
             load_sems, rs_send_sems, rs_recv_sems, ag_send_sems,
             ag_recv_sems):
        my = lax.axis_index("i")

        rs_buf[...] = jnp.zeros_like(rs_buf)

        wq_dma = pltpu.make_async_copy(
            wq_ref.at[:, pl.ds(my * d_loc, d_loc)], wq_s, load_sems.at[0])
        wq_dma.start()
        wo_dma = pltpu.make_async_copy(
            wo_ref.at[pl.ds(my * d_loc, d_loc), :], wo_s, load_sems.at[1])
        wo_dma.start()

        bar = pltpu.get_barrier_semaphore()
        for j in range(N_DEV):
            pl.semaphore_signal(bar, inc=1, device_id=(j,),
                                device_id_type=pl.DeviceIdType.MESH)
        pl.semaphore_wait(bar, N_DEV)

        wq_dma.wait()
        wo_dma.wait()

        xb = x_ref[...].astype(jnp.bfloat16).reshape(B * Sq, D)
        q = jnp.dot(xb, wq_s[...].astype(jnp.bfloat16),
                    preferred_element_type=jnp.float32)
        q_s[...] = (q * 0.125).reshape(B, Sq, d_loc).astype(jnp.bfloat16)

        qi = lax.broadcasted_iota(jnp.int32, (Sq, Skv), 0)
        ki = lax.broadcasted_iota(jnp.int32, (Sq, Skv), 1)
        mask = jnp.abs(qi - ki) <= WINDOW
        ones_col = jnp.ones((Skv, 1), jnp.bfloat16)

        for b in range(B):
            for h in range(H_loc):
                qbh = q_s[b, :, h * Dh:(h + 1) * Dh]
                kbh = k_ref[b, :, h, :].astype(jnp.bfloat16)
                s = lax.dot_general(
                    qbh, kbh, (((1,), (1,)), ((), ())),
                    preferred_element_type=jnp.float32)
                e = jnp.where(mask, jnp.exp(s), 0.0).astype(jnp.bfloat16)
                vbh = v_ref[b, :, h, :].astype(jnp.bfloat16)
                ctx_aug = jnp.dot(e, jnp.concatenate([vbh, ones_col], axis=1),
                                  preferred_element_type=jnp.float32)
                ctx_s[b, :, h * Dh:(h + 1) * Dh] = (
                    ctx_aug[:, :Dh] / ctx_aug[:, Dh:Dh + 1]
                ).astype(jnp.bfloat16)

            part = jnp.dot(ctx_s[b], wo_s[...].astype(jnp.bfloat16),
                           preferred_element_type=jnp.float32)
            acc_s[b] = part.astype(jnp.bfloat16)

            for j in range(N_DEV):
                @pl.when(my != j)
                def _(j=j, b=b):
                    pltpu.make_async_remote_copy(
                        src_ref=acc_s.at[pl.ds(b, 1), pl.ds(j * CH, CH), :],
                        dst_ref=rs_buf.at[pl.ds(b * N_DEV + my, 1)],
                        send_sem=rs_send_sems.at[b],
                        recv_sem=rs_recv_sems.at[b],
                        device_id=(j,),
                        device_id_type=pl.DeviceIdType.MESH,
                    ).start()

        for b in range(B):
            for j in range(N_DEV):
                @pl.when(my != j)
                def _(j=j, b=b):
                    pltpu.make_async_remote_copy(
                        src_ref=acc_s.at[pl.ds(b, 1), pl.ds(j * CH, CH), :],
                        dst_ref=rs_buf.at[pl.ds(b * N_DEV + j, 1)],
                        send_sem=rs_send_sems.at[b],
                        recv_sem=rs_recv_sems.at[b],
                        device_id=(j,),
                        device_id_type=pl.DeviceIdType.MESH,
                    ).wait_recv()

            own = acc_s[b, pl.ds(my * CH, CH), :].astype(jnp.float32)
            red = own + jnp.sum(
                rs_buf[b * N_DEV:(b + 1) * N_DEV].astype(jnp.float32), axis=0)
            out_ref[b, pl.ds(my * CH, CH), :] = red.astype(jnp.bfloat16)

            for j in range(N_DEV):
                @pl.when(my != j)
                def _(j=j, b=b):
                    pltpu.make_async_remote_copy(
                        src_ref=out_ref.at[pl.ds(b, 1), pl.ds(my * CH, CH), :],
                        dst_ref=out_ref.at[pl.ds(b, 1), pl.ds(my * CH, CH), :],
                        send_sem=ag_send_sems.at[b],
                        recv_sem=ag_recv_sems.at[b],
                        device_id=(j,),
                        device_id_type=pl.DeviceIdType.MESH,
                    ).start()

        for b in range(B):
            for j in range(N_DEV):
                @pl.when(my != j)
                def _(j=j, b=b):
                    pltpu.make_async_remote_copy(
                        src_ref=out_ref.at[pl.ds(b, 1), pl.ds(j * CH, CH), :],
                        dst_ref=out_ref.at[pl.ds(b, 1), pl.ds(j * CH, CH), :],
                        send_sem=ag_send_sems.at[b],
                        recv_sem=ag_recv_sems.at[b],
                        device_id=(j,),
                        device_id_type=pl.DeviceIdType.MESH,
                    ).wait_recv()

        for b in range(B):
            for j in range(N_DEV):
                @pl.when(my != j)
                def _(j=j, b=b):
                    pltpu.make_async_remote_copy(
                        src_ref=acc_s.at[pl.ds(b, 1), pl.ds(j * CH, CH), :],
                        dst_ref=rs_buf.at[pl.ds(b * N_DEV + j, 1)],
                        send_sem=rs_send_sems.at[b],
                        recv_sem=rs_recv_sems.at[b],
                        device_id=(j,),
                        device_id_type=pl.DeviceIdType.MESH,
                    ).wait_send()
                    pltpu.make_async_remote_copy(
                        src_ref=out_ref.at[pl.ds(b, 1), pl.ds(my * CH, CH), :],
                        dst_ref=out_ref.at[pl.ds(b, 1), pl.ds(my * CH, CH), :],
                        send_sem=ag_send_sems.at[b],
                        recv_sem=ag_recv_sems.at[b],
                        device_id=(j,),
                        device_id_type=pl.DeviceIdType.MESH,
                    ).wait_send()

    return pl.pallas_call(
        body,
        out_shape=jax.ShapeDtypeStruct((B, Sq, D), jnp.bfloat16),
        in_specs=[
            pl.BlockSpec(memory_space=pltpu.MemorySpace.VMEM),
            pl.BlockSpec(memory_space=pltpu.MemorySpace.HBM),
            pl.BlockSpec(memory_space=pltpu.MemorySpace.VMEM),
            pl.BlockSpec(memory_space=pltpu.MemorySpace.VMEM),
            pl.BlockSpec(memory_space=pltpu.MemorySpace.HBM),
        ],
        out_specs=pl.BlockSpec(memory_space=pltpu.MemorySpace.VMEM),
        scratch_shapes=[
            pltpu.VMEM((D, d_loc), jnp.float32),
            pltpu.VMEM((d_loc, D), jnp.float32),
            pltpu.VMEM((B, Sq, d_loc), jnp.bfloat16),
            pltpu.VMEM((B, Sq, d_loc), jnp.bfloat16),
            pltpu.VMEM((B, Sq, D), jnp.bfloat16),
            pltpu.VMEM((B * N_DEV, CH, D), jnp.bfloat16),
            pltpu.SemaphoreType.DMA((2,)),
            pltpu.SemaphoreType.DMA((2,)),
            pltpu.SemaphoreType.DMA((2,)),
            pltpu.SemaphoreType.DMA((2,)),
            pltpu.SemaphoreType.DMA((2,)),
        ],
        compiler_params=pltpu.CompilerParams(collective_id=0),
    )(x, Wq, K_ext, V_ext, Wo)
